# baseline (speedup 1.0000x reference)
import math

import numpy as np
import jax
import jax.numpy as jnp
from jax.experimental import pallas as pl
from jax.experimental.pallas import tpu as pltpu

_RATES = (1, 2)


def _bilinear_matrix(out_size, in_size):
    A = np.zeros((out_size, in_size), np.float32)
    if in_size == 1:
        A[:, 0] = 1.0
        return A
    scale = in_size / out_size
    for d in range(out_size):
        s = max((d + 0.5) * scale - 0.5, 0.0)
        i0 = min(int(math.floor(s)), in_size - 1)
        i1 = min(i0 + 1, in_size - 1)
        f = s - i0
        A[d, i0] += 1.0 - f
        A[d, i1] += f
    return A


def _rowshift(v, d):
    if d == 0:
        return v
    if d > 0:
        return jnp.concatenate(
            [v[d:], jnp.zeros((d, v.shape[1]), v.dtype)], axis=0)
    return jnp.concatenate(
        [jnp.zeros((-d, v.shape[1]), v.dtype), v[:d]], axis=0)


def _colshift(v, dw, colv, W):
    s = _rowshift(v, dw)
    zero = jnp.zeros((), v.dtype)
    if dw > 0:
        s = jnp.where(colv < (W - dw), s, zero)
    elif dw < 0:
        s = jnp.where(colv >= (-dw), s, zero)
    return s


def _blockT(x_t, colv, H, W, eye_bf, out_ct,
            w3pre, bpre_t, da, db, bdw_t, wpost, bpost_t, bpost_c):
    HW, C = x_t.shape
    cs = {dw: _colshift(x_t, dw, colv, W) for dw in (-1, 0, 1)}
    m3 = jnp.concatenate([cs[-1], cs[0], cs[1]], axis=1)
    p = jnp.dot(m3, w3pre, preferred_element_type=jnp.float32)
    x1 = (_rowshift(p[:, :C], -W) + p[:, C:2 * C] +
          _rowshift(p[:, 2 * C:], W))
    x1 = jnp.maximum(x1 + bpre_t, 0.0)
    x1b = x1.astype(jnp.bfloat16)

    csb = {dw: _colshift(x1b, dw, colv, W) for dw in (-2, -1, 1, 2)}
    m3a = jnp.concatenate([csb[-1], x1b, csb[1]], axis=1)
    m3b = jnp.concatenate([csb[-2], x1b, csb[2]], axis=1)
    pa = jnp.dot(m3a, da, preferred_element_type=jnp.float32)
    pb = jnp.dot(m3b, db, preferred_element_type=jnp.float32)
    b0 = (_rowshift(pa[:, :C], -W) + pa[:, C:2 * C] +
          _rowshift(pa[:, 2 * C:], W))
    b1 = (_rowshift(pb[:, :C], -2 * W) + pb[:, C:2 * C] +
          _rowshift(pb[:, 2 * C:], 2 * W))
    b0 = jnp.maximum(b0 + bdw_t[0:1, :], 0.0).astype(jnp.bfloat16)
    b1 = jnp.maximum(b1 + bdw_t[1:2, :], 0.0).astype(jnp.bfloat16)
    aspp = jnp.concatenate([b0, b1], axis=1)

    if out_ct:
        y = jax.lax.dot_general(
            wpost, aspp, (((1,), (1,)), ((), ())),
            preferred_element_type=jnp.float32)
        y = jnp.maximum(y + bpost_c, 0.0)
        x1_ct = jax.lax.dot_general(
            eye_bf, x1b, (((1,), (1,)), ((), ())),
            preferred_element_type=jnp.float32)
        return x1_ct + y
    y = jax.lax.dot_general(
        aspp, wpost, (((1,), (1,)), ((), ())),
        preferred_element_type=jnp.float32)
    y = jnp.maximum(y + bpost_t, 0.0)
    return x1 + y


def kernel(inner0_w_pre, inner0_b_pre, inner0_w_dw, inner0_b_dw,
           inner0_w_post, inner0_b_post,
           inner1_w_pre, inner1_b_pre, inner1_w_dw, inner1_b_dw,
           inner1_w_post, inner1_b_post,
           layer0_w_pre, layer0_b_pre, layer0_w_dw, layer0_b_dw,
           layer0_w_post, layer0_b_post,
           layer1_w_pre, layer1_b_pre, layer1_w_dw, layer1_b_dw,
           layer1_w_post, layer1_b_post,
           x0, x1):
    N, C, H0, W0 = x0.shape
    _, _, H1, W1 = x1.shape
    HW0, HW1 = H0 * W0, H1 * W1
    Cout = inner0_w_pre.shape[0]
    bf = jnp.bfloat16

    x0f = x0.reshape(N, C, HW0).astype(bf)
    x1f = x1.reshape(N, C, HW1).astype(bf)
    bkron = jnp.asarray(
        np.kron(_bilinear_matrix(H0, H1), _bilinear_matrix(W0, W1))
    ).astype(bf)

    eye_f = jnp.eye(C, dtype=jnp.float32)

    def diagblocks(wdw, r):
        rows = []
        for dwi in range(3):
            rows.append(jnp.concatenate(
                [eye_f * wdw[:, r * 9 + dhi * 3 + dwi][:, None]
                 for dhi in range(3)], axis=1))
        return jnp.concatenate(rows, axis=0).astype(bf)

    def prep(wpre, bpre, wdw, bdw, wpost, bpost):
        w3 = (wpre.reshape(C, 3, 3, C).transpose(2, 3, 1, 0)
              .reshape(3 * C, 3 * C).astype(bf))
        return (w3, bpre.T, diagblocks(wdw, 0), diagblocks(wdw, 1),
                bdw.T, wpost.astype(bf), bpost.T, bpost)

    p_i0 = prep(inner0_w_pre, inner0_b_pre, inner0_w_dw, inner0_b_dw,
                inner0_w_post, inner0_b_post)
    p_i1 = prep(inner1_w_pre, inner1_b_pre, inner1_w_dw, inner1_b_dw,
                inner1_w_post, inner1_b_post)
    p_l0 = prep(layer0_w_pre, layer0_b_pre, layer0_w_dw, layer0_b_dw,
                layer0_w_post, layer0_b_post)
    p_l1 = prep(layer1_w_pre, layer1_b_pre, layer1_w_dw, layer1_b_dw,
                layer1_w_post, layer1_b_post)

    def body(x0_ref, x1_ref, mt_ref,
             i0a, i0b, i0c, i0d, i0e, i0f, i0g, i0h,
             i1a, i1b, i1c, i1d, i1e, i1f, i1g, i1h,
             l0a, l0b, l0c, l0d, l0e, l0f, l0g, l0h,
             l1a, l1b, l1c, l1d, l1e, l1f, l1g, l1h,
             out0_ref, out1_ref):
        eye_bf = (jax.lax.broadcasted_iota(jnp.int32, (C, C), 0) ==
                  jax.lax.broadcasted_iota(jnp.int32, (C, C), 1)
                  ).astype(bf)
        colv1 = jax.lax.broadcasted_iota(jnp.int32, (HW1, 1), 0) % W1
        colv0 = jax.lax.broadcasted_iota(jnp.int32, (HW0, 1), 0) % W0

        def to_t(x_cf):
            return jax.lax.dot_general(
                x_cf, eye_bf, (((0,), (0,)), ((), ())),
                preferred_element_type=jnp.float32).astype(bf)

        def run(x_t, colv, H, W, ps, out_ct):
            return _blockT(x_t, colv, H, W, eye_bf, out_ct,
                           ps[0][...], ps[1][...], ps[2][...], ps[3][...],
                           ps[4][...], ps[5][...], ps[6][...], ps[7][...])

        x1t = to_t(x1_ref[0])
        t1 = run(x1t, colv1, H1, W1,
                 (i1a, i1b, i1c, i1d, i1e, i1f, i1g, i1h), False)
        t1_bf = t1.astype(bf)
        out1_ref[0] = run(t1_bf, colv1, H1, W1,
                          (l1a, l1b, l1c, l1d, l1e, l1f, l1g, l1h), True)

        x0t = to_t(x0_ref[0])
        lat = run(x0t, colv0, H0, W0,
                  (i0a, i0b, i0c, i0d, i0e, i0f, i0g, i0h), False)
        up = jnp.dot(mt_ref[...], t1_bf,
                     preferred_element_type=jnp.float32)
        merged = (lat + up).astype(bf)
        out0_ref[0] = run(merged, colv0, H0, W0,
                          (l0a, l0b, l0c, l0d, l0e, l0f, l0g, l0h), True)

    wspec = lambda shape: pl.BlockSpec(shape, lambda n: (0,) * len(shape))
    pspecs = []
    for ps in (p_i0, p_i1, p_l0, p_l1):
        pspecs += [wspec(a.shape) for a in ps]

    def call(x0s, x1s, *weights):
        nloc = x0s.shape[0]
        return pl.pallas_call(
            body,
            out_shape=(jax.ShapeDtypeStruct((nloc, Cout, HW0), x0.dtype),
                       jax.ShapeDtypeStruct((nloc, Cout, HW1), x0.dtype)),
            grid=(nloc,),
            in_specs=[
                pl.BlockSpec((1, C, HW0), lambda n: (n, 0, 0)),
                pl.BlockSpec((1, C, HW1), lambda n: (n, 0, 0)),
                wspec(bkron.shape),
            ] + pspecs,
            out_specs=(pl.BlockSpec((1, Cout, HW0), lambda n: (n, 0, 0)),
                       pl.BlockSpec((1, Cout, HW1), lambda n: (n, 0, 0))),
            compiler_params=pltpu.CompilerParams(
                dimension_semantics=("parallel",),
                vmem_limit_bytes=100 * 1024 * 1024,
            ),
        )(x0s, x1s, *weights)

    weights = (bkron, *p_i0, *p_i1, *p_l0, *p_l1)
    ndev = jax.local_device_count()
    if ndev > 1 and N % ndev == 0:
        mesh = jax.make_mesh((ndev,), ("d",))
        P = jax.sharding.PartitionSpec
        NS = jax.sharding.NamedSharding
        x0s = jax.reshard(x0f, NS(mesh, P("d", None, None)))
        x1s = jax.reshard(x1f, NS(mesh, P("d", None, None)))
        weights = tuple(
            jax.reshard(w, NS(mesh, P(*(None,) * w.ndim)))
            for w in weights)
        f = jax.shard_map(
            call, mesh=mesh, check_vma=False,
            in_specs=(P("d", None, None), P("d", None, None))
            + tuple(P(*(None,) * w.ndim) for w in weights),
            out_specs=(P("d", None, None), P("d", None, None)))
        out0, out1 = f(x0s, x1s, *weights)
    else:
        out0, out1 = call(x0f, x1f, *weights)

    return (out0.reshape(N, Cout, H0, W0), out1.reshape(N, Cout, H1, W1))


# speedup vs baseline: 4.2425x; 4.2425x over previous
import math

import numpy as np
import jax
import jax.numpy as jnp
from jax.experimental import pallas as pl
from jax.experimental.pallas import tpu as pltpu

_RATES = (1, 2)


def _bilinear_matrix(out_size, in_size):
    A = np.zeros((out_size, in_size), np.float32)
    if in_size == 1:
        A[:, 0] = 1.0
        return A
    scale = in_size / out_size
    for d in range(out_size):
        s = max((d + 0.5) * scale - 0.5, 0.0)
        i0 = min(int(math.floor(s)), in_size - 1)
        i1 = min(i0 + 1, in_size - 1)
        f = s - i0
        A[d, i0] += 1.0 - f
        A[d, i1] += f
    return A


def _rowshift(v, d):
    if d == 0:
        return v
    if d > 0:
        return jnp.concatenate(
            [v[d:], jnp.zeros((d, v.shape[1]), v.dtype)], axis=0)
    return jnp.concatenate(
        [jnp.zeros((-d, v.shape[1]), v.dtype), v[:d]], axis=0)


def _colshift(v, dw, colv, W):
    s = _rowshift(v, dw)
    zero = jnp.zeros((), v.dtype)
    if dw > 0:
        s = jnp.where(colv < (W - dw), s, zero)
    elif dw < 0:
        s = jnp.where(colv >= (-dw), s, zero)
    return s


def _blockT(x_t, colv, H, W, eye_bf, out_ct,
            w3pre, bpre_t, da, db, bdw_t, wpost, bpost_t, bpost_c):
    HW, C = x_t.shape
    cs = {dw: _colshift(x_t, dw, colv, W) for dw in (-1, 0, 1)}
    m3 = jnp.concatenate([cs[-1], cs[0], cs[1]], axis=1)
    p = jnp.dot(m3, w3pre, preferred_element_type=jnp.float32)
    x1 = (_rowshift(p[:, :C], -W) + p[:, C:2 * C] +
          _rowshift(p[:, 2 * C:], W))
    x1 = jnp.maximum(x1 + bpre_t, 0.0)
    x1b = x1.astype(jnp.bfloat16)

    csb = {dw: _colshift(x1b, dw, colv, W) for dw in (-2, -1, 1, 2)}
    m3a = jnp.concatenate([csb[-1], x1b, csb[1]], axis=1)
    m3b = jnp.concatenate([csb[-2], x1b, csb[2]], axis=1)
    pa = jnp.dot(m3a, da, preferred_element_type=jnp.float32)
    pb = jnp.dot(m3b, db, preferred_element_type=jnp.float32)
    b0 = (_rowshift(pa[:, :C], -W) + pa[:, C:2 * C] +
          _rowshift(pa[:, 2 * C:], W))
    b1 = (_rowshift(pb[:, :C], -2 * W) + pb[:, C:2 * C] +
          _rowshift(pb[:, 2 * C:], 2 * W))
    b0 = jnp.maximum(b0 + bdw_t[0:1, :], 0.0).astype(jnp.bfloat16)
    b1 = jnp.maximum(b1 + bdw_t[1:2, :], 0.0).astype(jnp.bfloat16)
    aspp = jnp.concatenate([b0, b1], axis=1)

    if out_ct:
        y = jax.lax.dot_general(
            wpost, aspp, (((1,), (1,)), ((), ())),
            preferred_element_type=jnp.float32)
        y = jnp.maximum(y + bpost_c, 0.0)
        x1_ct = jax.lax.dot_general(
            eye_bf, x1b, (((1,), (1,)), ((), ())),
            preferred_element_type=jnp.float32)
        return x1_ct + y
    y = jax.lax.dot_general(
        aspp, wpost, (((1,), (1,)), ((), ())),
        preferred_element_type=jnp.float32)
    y = jnp.maximum(y + bpost_t, 0.0)
    return x1 + y


def kernel(inner0_w_pre, inner0_b_pre, inner0_w_dw, inner0_b_dw,
           inner0_w_post, inner0_b_post,
           inner1_w_pre, inner1_b_pre, inner1_w_dw, inner1_b_dw,
           inner1_w_post, inner1_b_post,
           layer0_w_pre, layer0_b_pre, layer0_w_dw, layer0_b_dw,
           layer0_w_post, layer0_b_post,
           layer1_w_pre, layer1_b_pre, layer1_w_dw, layer1_b_dw,
           layer1_w_post, layer1_b_post,
           x0, x1):
    N, C, H0, W0 = x0.shape
    _, _, H1, W1 = x1.shape
    HW0, HW1 = H0 * W0, H1 * W1
    Cout = inner0_w_pre.shape[0]
    bf = jnp.bfloat16

    x0f = x0.reshape(N, C, HW0)
    x1f = x1.reshape(N, C, HW1)
    bkron = jnp.asarray(
        np.kron(_bilinear_matrix(H0, H1), _bilinear_matrix(W0, W1))
    ).astype(bf)

    wpres = jnp.stack(
        [inner1_w_pre, layer1_w_pre, inner0_w_pre, layer0_w_pre])
    wdws = jnp.stack([inner1_w_dw, layer1_w_dw, inner0_w_dw, layer0_w_dw])
    bpres = jnp.stack(
        [inner1_b_pre, layer1_b_pre, inner0_b_pre, layer0_b_pre])
    bdws = jnp.stack([inner1_b_dw, layer1_b_dw, inner0_b_dw, layer0_b_dw])
    wposts = jnp.stack(
        [inner1_w_post, layer1_w_post, inner0_w_post, layer0_w_post])
    bposts = jnp.stack(
        [inner1_b_post, layer1_b_post, inner0_b_post, layer0_b_post])

    w3s = (wpres.reshape(4, C, 3, 3, C).transpose(0, 3, 4, 2, 1)
           .reshape(4, 3 * C, 3 * C).astype(bf))
    eye_f = jnp.eye(C, dtype=jnp.float32)
    dadbs = (wdws.reshape(4, C, 2, 3, 3).transpose(0, 2, 4, 1, 3)
             [:, :, :, :, :, None] * eye_f[None, None, None, :, None, :]
             ).reshape(4, 2, 3 * C, 3 * C).astype(bf)
    wposts = wposts.astype(bf)
    bpres_t = bpres.reshape(4, 1, C)
    bdws_t = bdws.transpose(0, 2, 1)
    bposts_t = bposts.reshape(4, 1, C)

    def body(x0_ref, x1_ref, mt_ref, w3s_ref, dadbs_ref, wposts_ref,
             bpres_ref, bdws_ref, bpostst_ref, bpostsc_ref,
             out0_ref, out1_ref):
        eye_bf = (jax.lax.broadcasted_iota(jnp.int32, (C, C), 0) ==
                  jax.lax.broadcasted_iota(jnp.int32, (C, C), 1)
                  ).astype(bf)
        colv1 = jax.lax.broadcasted_iota(jnp.int32, (HW1, 1), 0) % W1
        colv0 = jax.lax.broadcasted_iota(jnp.int32, (HW0, 1), 0) % W0

        def to_t(x_cf):
            return jax.lax.dot_general(
                x_cf.astype(bf), eye_bf, (((0,), (0,)), ((), ())),
                preferred_element_type=jnp.float32).astype(bf)

        def run(x_t, colv, H, W, s, out_ct):
            return _blockT(x_t, colv, H, W, eye_bf, out_ct,
                           w3s_ref[s], bpres_ref[s], dadbs_ref[s, 0],
                           dadbs_ref[s, 1], bdws_ref[s], wposts_ref[s],
                           bpostst_ref[s], bpostsc_ref[s])

        x1t = to_t(x1_ref[0])
        t1 = run(x1t, colv1, H1, W1, 0, False)
        t1_bf = t1.astype(bf)
        out1_ref[0] = run(t1_bf, colv1, H1, W1, 1, True)

        x0t = to_t(x0_ref[0])
        lat = run(x0t, colv0, H0, W0, 2, False)
        up = jnp.dot(mt_ref[...], t1_bf,
                     preferred_element_type=jnp.float32)
        merged = (lat + up).astype(bf)
        out0_ref[0] = run(merged, colv0, H0, W0, 3, True)

    wspec = lambda shape: pl.BlockSpec(shape, lambda n: (0,) * len(shape))
    weights = (bkron, w3s, dadbs, wposts, bpres_t, bdws_t, bposts_t,
               bposts)

    out0, out1 = pl.pallas_call(
        body,
        out_shape=(jax.ShapeDtypeStruct((N, Cout, HW0), x0.dtype),
                   jax.ShapeDtypeStruct((N, Cout, HW1), x0.dtype)),
        grid=(N,),
        in_specs=[
            pl.BlockSpec((1, C, HW0), lambda n: (n, 0, 0)),
            pl.BlockSpec((1, C, HW1), lambda n: (n, 0, 0)),
        ] + [wspec(w.shape) for w in weights],
        out_specs=(pl.BlockSpec((1, Cout, HW0), lambda n: (n, 0, 0)),
                   pl.BlockSpec((1, Cout, HW1), lambda n: (n, 0, 0))),
        compiler_params=pltpu.CompilerParams(
            dimension_semantics=("parallel",),
            vmem_limit_bytes=100 * 1024 * 1024,
        ),
    )(x0f, x1f, *weights)

    return (out0.reshape(N, Cout, H0, W0), out1.reshape(N, Cout, H1, W1))
